# Initial kernel scaffold; baseline (speedup 1.0000x reference)
import jax
import jax.numpy as jnp
from jax.experimental import pallas as pl


def kernel(queries, keys, k):
    raise NotImplementedError("write your pallas kernel here")


# speedup vs baseline: 2.5403x; 2.5403x over previous
import functools

import jax
import jax.numpy as jnp
from jax import lax
from jax.experimental import pallas as pl
from jax.experimental.pallas import tpu as pltpu
from jax.experimental.pallas import tpu_sc as plsc

_Q = 1024
_D = 128
_NKEYS = 100000
_BK = 2048
_KB = 49
_TOPK = 3
_BIG_IDX = 2**30


def _build_topk_body(nkeys, bk, kb_total, topk):
    def body(q_ref, k_ref, dist_ref, idx_ref, d2s_ref, idxs_ref):
        kb = pl.program_id(0)
        q = q_ref[...]
        kt = k_ref[...]
        qk = lax.dot_general(q, kt, (((1,), (1,)), ((), ())),
                             preferred_element_type=jnp.float32)
        q_sq = jnp.sum(q * q, axis=1, keepdims=True)
        k_sq = jnp.sum(kt * kt, axis=1)[None, :]
        d2 = jnp.maximum(q_sq + k_sq - 2.0 * qk, 0.0)
        gidx = kb * bk + lax.broadcasted_iota(jnp.int32, d2.shape, 1)
        d2 = jnp.where(gidx < nkeys, d2, jnp.inf)

        vals, idxs = [], []
        for t in range(topk):
            m = jnp.min(d2, axis=1, keepdims=True)
            sel = jnp.min(jnp.where(d2 == m, gidx, _BIG_IDX), axis=1,
                          keepdims=True)
            if t < topk - 1:
                d2 = jnp.where(gidx == sel, jnp.inf, d2)
            vals.append(m)
            idxs.append(sel)
        td2 = jnp.concatenate(vals, axis=1)
        tidx = jnp.concatenate(idxs, axis=1)

        @pl.when(kb == 0)
        def _():
            d2s_ref[...] = td2
            idxs_ref[...] = tidx

        @pl.when(kb > 0)
        def _():
            d2c = jnp.concatenate([d2s_ref[...], td2], axis=1)
            idxc = jnp.concatenate([idxs_ref[...], tidx], axis=1)
            nv, ni = [], []
            for _t in range(topk):
                m = jnp.min(d2c, axis=1, keepdims=True)
                sel = jnp.min(jnp.where(d2c == m, idxc, _BIG_IDX), axis=1,
                              keepdims=True)
                d2c = jnp.where((d2c == m) & (idxc == sel), jnp.inf, d2c)
                nv.append(m)
                ni.append(sel)
            d2s_ref[...] = jnp.concatenate(nv, axis=1)
            idxs_ref[...] = jnp.concatenate(ni, axis=1)

        @pl.when(kb == kb_total - 1)
        def _():
            dist_ref[...] = jnp.sqrt(d2s_ref[...])
            idx_ref[...] = idxs_ref[...]

    return body


def _topk_call(queries, keys_padded):
    q, d = queries.shape
    return pl.pallas_call(
        _build_topk_body(_NKEYS, _BK, _KB, _TOPK),
        grid=(_KB,),
        in_specs=[
            pl.BlockSpec((q, d), lambda kb: (0, 0)),
            pl.BlockSpec((_BK, d), lambda kb: (kb, 0)),
        ],
        out_specs=[
            pl.BlockSpec((q, _TOPK), lambda kb: (0, 0)),
            pl.BlockSpec((q, _TOPK), lambda kb: (0, 0)),
        ],
        out_shape=[
            jax.ShapeDtypeStruct((q, _TOPK), jnp.float32),
            jax.ShapeDtypeStruct((q, _TOPK), jnp.int32),
        ],
        scratch_shapes=[
            pltpu.VMEM((q, _TOPK), jnp.float32),
            pltpu.VMEM((q, _TOPK), jnp.int32),
        ],
    )(queries, keys_padded)


def _sc_gather(keys, flat_idx):
    b, d = flat_idx.shape[0], keys.shape[1]
    info = plsc.get_sparse_core_info()
    nw = info.num_cores * info.num_subcores
    bpw = b // nw
    mesh = plsc.VectorSubcoreMesh(core_axis_name="c", subcore_axis_name="s")

    @functools.partial(
        pl.kernel,
        mesh=mesh,
        out_type=jax.ShapeDtypeStruct((b, d), jnp.float32),
        scratch_types=[
            pltpu.VMEM((bpw,), jnp.int32),
            pltpu.VMEM((bpw, d), jnp.float32),
            pltpu.SemaphoreType.DMA,
        ],
    )
    def gather_kernel(keys_hbm, idx_hbm, out_hbm, idx_v, rows_v, sem):
        wid = lax.axis_index("s") * info.num_cores + lax.axis_index("c")
        base = wid * bpw
        pltpu.sync_copy(idx_hbm.at[pl.ds(base, bpw)], idx_v)
        pltpu.async_copy(keys_hbm.at[idx_v], rows_v, sem).wait()
        pltpu.sync_copy(rows_v, out_hbm.at[pl.ds(base, bpw)])

    return gather_kernel(keys, flat_idx)


def kernel(queries, keys, k):
    del k
    keys_padded = jnp.pad(keys, ((0, _KB * _BK - _NKEYS), (0, 0)))
    topk_dist, topk_idx = _topk_call(queries, keys_padded)
    gathered = _sc_gather(keys, topk_idx.reshape(-1))
    return gathered.reshape(_Q, _TOPK, _D), topk_dist, topk_idx


# speedup vs baseline: 3.4184x; 1.3457x over previous
import functools

import jax
import jax.numpy as jnp
from jax import lax
from jax.experimental import pallas as pl
from jax.experimental.pallas import tpu as pltpu
from jax.experimental.pallas import tpu_sc as plsc

_Q = 1024
_D = 128
_NKEYS = 100000
_BK = 4096
_KB = 25
_TOPK = 3
_BIG_IDX = 2**30


def _build_topk_body(nkeys, bk, kb_total, topk):
    big_f = float(2.0 ** 30)

    def body(q_ref, k_ref, dist_ref, idx_ref, d2s_ref, idxs_ref):
        kb = pl.program_id(0)
        q = q_ref[...]
        kt = k_ref[...]
        qk = lax.dot_general(q, kt, (((1,), (1,)), ((), ())),
                             preferred_element_type=jnp.float32)
        q_sq = jnp.sum(q * q, axis=1, keepdims=True)
        k_sq = jnp.sum(kt * kt, axis=1)[None, :]
        lane = lax.broadcasted_iota(jnp.int32, (1, bk), 1)
        k_sq = jnp.where(kb * bk + lane < nkeys, k_sq, jnp.inf)
        d2 = jnp.maximum((q_sq + k_sq) - 2.0 * qk, 0.0)
        iotaf = lax.broadcasted_iota(jnp.int32, d2.shape, 1).astype(jnp.float32)

        vals, idxs = [], []
        for t in range(topk):
            m = jnp.min(d2, axis=1, keepdims=True)
            sel = jnp.min(jnp.where(d2 == m, iotaf, big_f), axis=1,
                          keepdims=True)
            if t < topk - 1:
                d2 = jnp.where(iotaf == sel, jnp.inf, d2)
            vals.append(m)
            idxs.append(sel)
        td2 = jnp.concatenate(vals, axis=1)
        tidx = jnp.float32(kb * bk) + jnp.concatenate(idxs, axis=1)

        @pl.when(kb == 0)
        def _():
            d2s_ref[...] = td2
            idxs_ref[...] = tidx

        @pl.when(kb > 0)
        def _():
            d2c = jnp.concatenate([d2s_ref[...], td2], axis=1)
            idxc = jnp.concatenate([idxs_ref[...], tidx], axis=1)
            nv, ni = [], []
            for _t in range(topk):
                m = jnp.min(d2c, axis=1, keepdims=True)
                sel = jnp.min(jnp.where(d2c == m, idxc, big_f), axis=1,
                              keepdims=True)
                if _t < topk - 1:
                    d2c = jnp.where(idxc == sel, jnp.inf, d2c)
                nv.append(m)
                ni.append(sel)
            d2s_ref[...] = jnp.concatenate(nv, axis=1)
            idxs_ref[...] = jnp.concatenate(ni, axis=1)

        @pl.when(kb == kb_total - 1)
        def _():
            dist_ref[...] = jnp.sqrt(d2s_ref[...])
            idx_ref[...] = idxs_ref[...].astype(jnp.int32)

    return body


def _topk_call(queries, keys_padded):
    q, d = queries.shape
    return pl.pallas_call(
        _build_topk_body(_NKEYS, _BK, _KB, _TOPK),
        grid=(_KB,),
        in_specs=[
            pl.BlockSpec((q, d), lambda kb: (0, 0)),
            pl.BlockSpec((_BK, d), lambda kb: (kb, 0)),
        ],
        out_specs=[
            pl.BlockSpec((q, _TOPK), lambda kb: (0, 0)),
            pl.BlockSpec((q, _TOPK), lambda kb: (0, 0)),
        ],
        out_shape=[
            jax.ShapeDtypeStruct((q, _TOPK), jnp.float32),
            jax.ShapeDtypeStruct((q, _TOPK), jnp.int32),
        ],
        scratch_shapes=[
            pltpu.VMEM((q, _TOPK), jnp.float32),
            pltpu.VMEM((q, _TOPK), jnp.float32),
        ],
    )(queries, keys_padded)


def _sc_gather(keys, flat_idx):
    b, d = flat_idx.shape[0], keys.shape[1]
    info = plsc.get_sparse_core_info()
    nw = info.num_cores * info.num_subcores
    bpw = b // nw
    mesh = plsc.VectorSubcoreMesh(core_axis_name="c", subcore_axis_name="s")

    @functools.partial(
        pl.kernel,
        mesh=mesh,
        out_type=jax.ShapeDtypeStruct((b, d), jnp.float32),
        scratch_types=[
            pltpu.VMEM((bpw,), jnp.int32),
            pltpu.VMEM((bpw, d), jnp.float32),
            pltpu.SemaphoreType.DMA,
        ],
    )
    def gather_kernel(keys_hbm, idx_hbm, out_hbm, idx_v, rows_v, sem):
        wid = lax.axis_index("s") * info.num_cores + lax.axis_index("c")
        base = wid * bpw
        pltpu.sync_copy(idx_hbm.at[pl.ds(base, bpw)], idx_v)
        pltpu.async_copy(keys_hbm.at[idx_v], rows_v, sem).wait()
        pltpu.sync_copy(rows_v, out_hbm.at[pl.ds(base, bpw)])

    return gather_kernel(keys, flat_idx)


def kernel(queries, keys, k):
    del k
    keys_padded = jnp.pad(keys, ((0, _KB * _BK - _NKEYS), (0, 0)))
    topk_dist, topk_idx = _topk_call(queries, keys_padded)
    gathered = _sc_gather(keys, topk_idx.reshape(-1))
    return gathered.reshape(_Q, _TOPK, _D), topk_dist, topk_idx


# speedup vs baseline: 3.6542x; 1.0690x over previous
import functools

import jax
import jax.numpy as jnp
from jax import lax
from jax.experimental import pallas as pl
from jax.experimental.pallas import tpu as pltpu
from jax.experimental.pallas import tpu_sc as plsc

_Q = 1024
_D = 128
_NKEYS = 100000
_BK = 4000
_KB = 25
_TOPK = 3
_BIG_IDX = 2**30


def _build_topk_body(nkeys, bk, kb_total, topk):
    big_f = float(2.0 ** 30)

    def body(q_ref, k_ref, dist_ref, idx_ref, d2s_ref, idxs_ref):
        kb = pl.program_id(0)
        q = q_ref[...]
        kt = k_ref[...]
        qk = lax.dot_general(q, kt, (((1,), (1,)), ((), ())),
                             preferred_element_type=jnp.float32)
        q_sq = jnp.sum(q * q, axis=1, keepdims=True)
        k_sq = jnp.sum(kt * kt, axis=1)[None, :]
        d2 = jnp.maximum((q_sq + k_sq) - 2.0 * qk, 0.0)
        iotaf = lax.broadcasted_iota(jnp.int32, d2.shape, 1).astype(jnp.float32)

        vals, idxs = [], []
        for t in range(topk):
            m = jnp.min(d2, axis=1, keepdims=True)
            sel = jnp.min(jnp.where(d2 == m, iotaf, big_f), axis=1,
                          keepdims=True)
            if t < topk - 1:
                d2 = jnp.where(iotaf == sel, jnp.inf, d2)
            vals.append(m)
            idxs.append(sel)
        td2 = jnp.concatenate(vals, axis=1)
        tidx = jnp.float32(kb * bk) + jnp.concatenate(idxs, axis=1)

        @pl.when(kb == 0)
        def _():
            d2s_ref[...] = td2
            idxs_ref[...] = tidx

        @pl.when(kb > 0)
        def _():
            d2c = jnp.concatenate([d2s_ref[...], td2], axis=1)
            idxc = jnp.concatenate([idxs_ref[...], tidx], axis=1)
            nv, ni = [], []
            for _t in range(topk):
                m = jnp.min(d2c, axis=1, keepdims=True)
                sel = jnp.min(jnp.where(d2c == m, idxc, big_f), axis=1,
                              keepdims=True)
                if _t < topk - 1:
                    d2c = jnp.where(idxc == sel, jnp.inf, d2c)
                nv.append(m)
                ni.append(sel)
            d2s_ref[...] = jnp.concatenate(nv, axis=1)
            idxs_ref[...] = jnp.concatenate(ni, axis=1)

        @pl.when(kb == kb_total - 1)
        def _():
            dist_ref[...] = jnp.sqrt(d2s_ref[...])
            idx_ref[...] = idxs_ref[...].astype(jnp.int32)

    return body


def _topk_call(queries, keys_padded):
    q, d = queries.shape
    return pl.pallas_call(
        _build_topk_body(_NKEYS, _BK, _KB, _TOPK),
        grid=(_KB,),
        in_specs=[
            pl.BlockSpec((q, d), lambda kb: (0, 0)),
            pl.BlockSpec((_BK, d), lambda kb: (kb, 0)),
        ],
        out_specs=[
            pl.BlockSpec((q, _TOPK), lambda kb: (0, 0)),
            pl.BlockSpec((q, _TOPK), lambda kb: (0, 0)),
        ],
        out_shape=[
            jax.ShapeDtypeStruct((q, _TOPK), jnp.float32),
            jax.ShapeDtypeStruct((q, _TOPK), jnp.int32),
        ],
        scratch_shapes=[
            pltpu.VMEM((q, _TOPK), jnp.float32),
            pltpu.VMEM((q, _TOPK), jnp.float32),
        ],
    )(queries, keys_padded)


def _sc_gather(keys, flat_idx):
    b, d = flat_idx.shape[0], keys.shape[1]
    info = plsc.get_sparse_core_info()
    nw = info.num_cores * info.num_subcores
    bpw = b // nw
    mesh = plsc.VectorSubcoreMesh(core_axis_name="c", subcore_axis_name="s")

    @functools.partial(
        pl.kernel,
        mesh=mesh,
        out_type=jax.ShapeDtypeStruct((b, d), jnp.float32),
        scratch_types=[
            pltpu.VMEM((bpw,), jnp.int32),
            pltpu.VMEM((bpw, d), jnp.float32),
            pltpu.SemaphoreType.DMA,
        ],
    )
    def gather_kernel(keys_hbm, idx_hbm, out_hbm, idx_v, rows_v, sem):
        wid = lax.axis_index("s") * info.num_cores + lax.axis_index("c")
        base = wid * bpw
        pltpu.sync_copy(idx_hbm.at[pl.ds(base, bpw)], idx_v)
        pltpu.async_copy(keys_hbm.at[idx_v], rows_v, sem).wait()
        pltpu.sync_copy(rows_v, out_hbm.at[pl.ds(base, bpw)])

    return gather_kernel(keys, flat_idx)


def kernel(queries, keys, k):
    del k
    topk_dist, topk_idx = _topk_call(queries, keys)
    gathered = _sc_gather(keys, topk_idx.reshape(-1))
    return gathered.reshape(_Q, _TOPK, _D), topk_dist, topk_idx


# speedup vs baseline: 3.8987x; 1.0669x over previous
import functools

import jax
import jax.numpy as jnp
from jax import lax
from jax.experimental import pallas as pl
from jax.experimental.pallas import tpu as pltpu
from jax.experimental.pallas import tpu_sc as plsc

_Q = 1024
_D = 128
_NKEYS = 100000
_BK = 4000
_KB = 25
_TOPK = 3
_BIG_IDX = 2**30


def _build_topk_body(nkeys, bk, kb_total, topk):
    big_f = float(2.0 ** 30)

    def body(q_ref, k_ref, dist_ref, idx_ref, d2s_ref, idxs_ref):
        kb = pl.program_id(0)
        q = q_ref[...]
        kt = k_ref[...]
        qk2 = lax.dot_general(q * -2.0, kt, (((1,), (1,)), ((), ())),
                              preferred_element_type=jnp.float32)
        q_sq = jnp.sum(q * q, axis=1, keepdims=True)
        k_sq = jnp.sum(kt * kt, axis=1)[None, :]
        d2 = jnp.maximum((q_sq + k_sq) + qk2, 0.0)
        iotaf = lax.broadcasted_iota(jnp.int32, d2.shape, 1).astype(jnp.float32)

        vals, idxs = [], []
        for t in range(topk):
            m = jnp.min(d2, axis=1, keepdims=True)
            sel = jnp.min(jnp.where(d2 == m, iotaf, big_f), axis=1,
                          keepdims=True)
            if t < topk - 1:
                d2 = jnp.where(iotaf == sel, jnp.inf, d2)
            vals.append(m)
            idxs.append(sel)
        td2 = jnp.concatenate(vals, axis=1)
        tidx = jnp.float32(kb * bk) + jnp.concatenate(idxs, axis=1)

        @pl.when(kb == 0)
        def _():
            d2s_ref[...] = td2
            idxs_ref[...] = tidx

        @pl.when(kb > 0)
        def _():
            d2c = jnp.concatenate([d2s_ref[...], td2], axis=1)
            idxc = jnp.concatenate([idxs_ref[...], tidx], axis=1)
            nv, ni = [], []
            for _t in range(topk):
                m = jnp.min(d2c, axis=1, keepdims=True)
                sel = jnp.min(jnp.where(d2c == m, idxc, big_f), axis=1,
                              keepdims=True)
                if _t < topk - 1:
                    d2c = jnp.where(idxc == sel, jnp.inf, d2c)
                nv.append(m)
                ni.append(sel)
            d2s_ref[...] = jnp.concatenate(nv, axis=1)
            idxs_ref[...] = jnp.concatenate(ni, axis=1)

        @pl.when(kb == kb_total - 1)
        def _():
            dist_ref[...] = jnp.sqrt(d2s_ref[...])
            idx_ref[...] = idxs_ref[...].astype(jnp.int32)

    return body


def _topk_call(queries, keys_padded):
    q, d = queries.shape
    return pl.pallas_call(
        _build_topk_body(_NKEYS, _BK, _KB, _TOPK),
        grid=(_KB,),
        in_specs=[
            pl.BlockSpec((q, d), lambda kb: (0, 0)),
            pl.BlockSpec((_BK, d), lambda kb: (kb, 0)),
        ],
        out_specs=[
            pl.BlockSpec((q, _TOPK), lambda kb: (0, 0)),
            pl.BlockSpec((q, _TOPK), lambda kb: (0, 0)),
        ],
        out_shape=[
            jax.ShapeDtypeStruct((q, _TOPK), jnp.float32),
            jax.ShapeDtypeStruct((q, _TOPK), jnp.int32),
        ],
        scratch_shapes=[
            pltpu.VMEM((q, _TOPK), jnp.float32),
            pltpu.VMEM((q, _TOPK), jnp.float32),
        ],
    )(queries, keys_padded)


def _sc_gather(keys, flat_idx):
    b, d = flat_idx.shape[0], keys.shape[1]
    info = plsc.get_sparse_core_info()
    nw = info.num_cores * info.num_subcores
    bpw = b // nw
    mesh = plsc.VectorSubcoreMesh(core_axis_name="c", subcore_axis_name="s")

    @functools.partial(
        pl.kernel,
        mesh=mesh,
        out_type=jax.ShapeDtypeStruct((b, d), jnp.float32),
        scratch_types=[
            pltpu.VMEM((bpw,), jnp.int32),
            pltpu.VMEM((bpw, d), jnp.float32),
            pltpu.SemaphoreType.DMA,
        ],
    )
    def gather_kernel(keys_hbm, idx_hbm, out_hbm, idx_v, rows_v, sem):
        wid = lax.axis_index("s") * info.num_cores + lax.axis_index("c")
        base = wid * bpw
        pltpu.sync_copy(idx_hbm.at[pl.ds(base, bpw)], idx_v)
        pltpu.async_copy(keys_hbm.at[idx_v], rows_v, sem).wait()
        pltpu.sync_copy(rows_v, out_hbm.at[pl.ds(base, bpw)])

    return gather_kernel(keys, flat_idx)


def kernel(queries, keys, k):
    del k
    topk_dist, topk_idx = _topk_call(queries, keys)
    gathered = _sc_gather(keys, topk_idx.reshape(-1))
    return gathered.reshape(_Q, _TOPK, _D), topk_dist, topk_idx


# speedup vs baseline: 4.0183x; 1.0307x over previous
import functools

import jax
import jax.numpy as jnp
from jax import lax
from jax.experimental import pallas as pl
from jax.experimental.pallas import tpu as pltpu
from jax.experimental.pallas import tpu_sc as plsc

_Q = 1024
_D = 128
_NKEYS = 100000
_BK = 5000
_KB = 20
_TOPK = 3
_BIG_IDX = 2**30


def _build_topk_body(nkeys, bk, kb_total, topk):
    big_f = float(2.0 ** 30)

    def body(q_ref, k_ref, dist_ref, idx_ref, d2s_ref, idxs_ref):
        kb = pl.program_id(0)
        q = q_ref[...]
        kt = k_ref[...]
        qk2 = lax.dot_general(q * -2.0, kt, (((1,), (1,)), ((), ())),
                              preferred_element_type=jnp.float32)
        q_sq = jnp.sum(q * q, axis=1, keepdims=True)
        k_sq = jnp.sum(kt * kt, axis=1)[None, :]
        d2 = jnp.maximum((q_sq + k_sq) + qk2, 0.0)
        iotaf = lax.broadcasted_iota(jnp.int32, d2.shape, 1).astype(jnp.float32)

        vals, idxs = [], []
        for t in range(topk):
            m = jnp.min(d2, axis=1, keepdims=True)
            sel = jnp.min(jnp.where(d2 == m, iotaf, big_f), axis=1,
                          keepdims=True)
            if t < topk - 1:
                d2 = jnp.where(iotaf == sel, jnp.inf, d2)
            vals.append(m)
            idxs.append(sel)
        td2 = jnp.concatenate(vals, axis=1)
        tidx = jnp.float32(kb * bk) + jnp.concatenate(idxs, axis=1)

        @pl.when(kb == 0)
        def _():
            d2s_ref[...] = td2
            idxs_ref[...] = tidx

        @pl.when(kb > 0)
        def _():
            d2c = jnp.concatenate([d2s_ref[...], td2], axis=1)
            idxc = jnp.concatenate([idxs_ref[...], tidx], axis=1)
            nv, ni = [], []
            for _t in range(topk):
                m = jnp.min(d2c, axis=1, keepdims=True)
                sel = jnp.min(jnp.where(d2c == m, idxc, big_f), axis=1,
                              keepdims=True)
                if _t < topk - 1:
                    d2c = jnp.where(idxc == sel, jnp.inf, d2c)
                nv.append(m)
                ni.append(sel)
            d2s_ref[...] = jnp.concatenate(nv, axis=1)
            idxs_ref[...] = jnp.concatenate(ni, axis=1)

        @pl.when(kb == kb_total - 1)
        def _():
            dist_ref[...] = jnp.sqrt(d2s_ref[...])
            idx_ref[...] = idxs_ref[...].astype(jnp.int32)

    return body


def _topk_call(queries, keys_padded):
    q, d = queries.shape
    return pl.pallas_call(
        _build_topk_body(_NKEYS, _BK, _KB, _TOPK),
        grid=(_KB,),
        in_specs=[
            pl.BlockSpec((q, d), lambda kb: (0, 0)),
            pl.BlockSpec((_BK, d), lambda kb: (kb, 0)),
        ],
        out_specs=[
            pl.BlockSpec((q, _TOPK), lambda kb: (0, 0)),
            pl.BlockSpec((q, _TOPK), lambda kb: (0, 0)),
        ],
        out_shape=[
            jax.ShapeDtypeStruct((q, _TOPK), jnp.float32),
            jax.ShapeDtypeStruct((q, _TOPK), jnp.int32),
        ],
        scratch_shapes=[
            pltpu.VMEM((q, _TOPK), jnp.float32),
            pltpu.VMEM((q, _TOPK), jnp.float32),
        ],
    )(queries, keys_padded)


def _sc_gather(keys, flat_idx):
    b, d = flat_idx.shape[0], keys.shape[1]
    info = plsc.get_sparse_core_info()
    nw = info.num_cores * info.num_subcores
    bpw = b // nw
    mesh = plsc.VectorSubcoreMesh(core_axis_name="c", subcore_axis_name="s")

    @functools.partial(
        pl.kernel,
        mesh=mesh,
        out_type=jax.ShapeDtypeStruct((b, d), jnp.float32),
        scratch_types=[
            pltpu.VMEM((bpw,), jnp.int32),
            pltpu.VMEM((bpw, d), jnp.float32),
            pltpu.SemaphoreType.DMA,
        ],
    )
    def gather_kernel(keys_hbm, idx_hbm, out_hbm, idx_v, rows_v, sem):
        wid = lax.axis_index("s") * info.num_cores + lax.axis_index("c")
        base = wid * bpw
        pltpu.sync_copy(idx_hbm.at[pl.ds(base, bpw)], idx_v)
        pltpu.async_copy(keys_hbm.at[idx_v], rows_v, sem).wait()
        pltpu.sync_copy(rows_v, out_hbm.at[pl.ds(base, bpw)])

    return gather_kernel(keys, flat_idx)


def kernel(queries, keys, k):
    del k
    topk_dist, topk_idx = _topk_call(queries, keys)
    gathered = _sc_gather(keys, topk_idx.reshape(-1))
    return gathered.reshape(_Q, _TOPK, _D), topk_dist, topk_idx


# speedup vs baseline: 4.0329x; 1.0036x over previous
import functools

import jax
import jax.numpy as jnp
from jax import lax
from jax.experimental import pallas as pl
from jax.experimental.pallas import tpu as pltpu
from jax.experimental.pallas import tpu_sc as plsc

_Q = 1024
_D = 128
_NKEYS = 100000
_BK = 5000
_KB = 20
_TOPK = 3
_BIG_IDX = 2**30


def _build_topk_body(nkeys, bk, kb_total, topk):
    big_f = float(2.0 ** 30)

    def tile_topk(d2, iotaf, base_f):
        vals, idxs = [], []
        for t in range(topk):
            m = jnp.min(d2, axis=1, keepdims=True)
            sel = jnp.min(jnp.where(d2 == m, iotaf, big_f), axis=1,
                          keepdims=True)
            if t < topk - 1:
                d2 = jnp.where(iotaf == sel, jnp.inf, d2)
            vals.append(m)
            idxs.append(sel)
        td2 = jnp.concatenate(vals, axis=1)
        tidx = base_f + jnp.concatenate(idxs, axis=1)
        return td2, tidx

    def merge(d2s, idxs, td2, tidx):
        d2c = jnp.concatenate([d2s, td2], axis=1)
        idxc = jnp.concatenate([idxs, tidx], axis=1)
        nv, ni = [], []
        for t in range(topk):
            m = jnp.min(d2c, axis=1, keepdims=True)
            sel = jnp.min(jnp.where(d2c == m, idxc, big_f), axis=1,
                          keepdims=True)
            if t < topk - 1:
                d2c = jnp.where(idxc == sel, jnp.inf, d2c)
            nv.append(m)
            ni.append(sel)
        return jnp.concatenate(nv, axis=1), jnp.concatenate(ni, axis=1)

    def body(q_ref, khbm_ref, dist_ref, idx_ref, kbuf_ref, sem_ref):
        def copy_in(j, slot):
            return pltpu.make_async_copy(
                khbm_ref.at[pl.ds(j * bk, bk), :], kbuf_ref.at[slot],
                sem_ref.at[slot])

        copy_in(0, 0).start()
        q = q_ref[...]
        qm2 = q * -2.0
        q_sq = jnp.sum(q * q, axis=1, keepdims=True)
        iotaf = lax.broadcasted_iota(
            jnp.int32, (q.shape[0], bk), 1).astype(jnp.float32)

        def process(j, slot, carry):
            d2s, idxs = carry
            @pl.when(j + 1 < kb_total)
            def _():
                copy_in(j + 1, 1 - slot).start()
            copy_in(j, slot).wait()
            kt = kbuf_ref[slot]
            qk2 = lax.dot_general(qm2, kt, (((1,), (1,)), ((), ())),
                                  preferred_element_type=jnp.float32)
            k_sq = jnp.sum(kt * kt, axis=1)[None, :]
            d2 = jnp.maximum((q_sq + k_sq) + qk2, 0.0)
            td2, tidx = tile_topk(d2, iotaf, jnp.float32(j * bk))
            return merge(d2s, idxs, td2, tidx)

        def step(i, carry):
            j = i * 2
            carry = process(j, 0, carry)
            return process(j + 1, 1, carry)

        d2s0 = jnp.full((q.shape[0], topk), jnp.inf, jnp.float32)
        idxs0 = jnp.full((q.shape[0], topk), big_f, jnp.float32)
        d2s, idxs = lax.fori_loop(0, kb_total // 2, step, (d2s0, idxs0))

        dist_ref[...] = jnp.sqrt(d2s)
        idx_ref[...] = idxs.astype(jnp.int32)

    return body


def _topk_call(queries, keys):
    q, d = queries.shape
    return pl.pallas_call(
        _build_topk_body(_NKEYS, _BK, _KB, _TOPK),
        in_specs=[
            pl.BlockSpec(memory_space=pltpu.VMEM),
            pl.BlockSpec(memory_space=pl.ANY),
        ],
        out_specs=[
            pl.BlockSpec(memory_space=pltpu.VMEM),
            pl.BlockSpec(memory_space=pltpu.VMEM),
        ],
        out_shape=[
            jax.ShapeDtypeStruct((q, _TOPK), jnp.float32),
            jax.ShapeDtypeStruct((q, _TOPK), jnp.int32),
        ],
        scratch_shapes=[
            pltpu.VMEM((2, _BK, d), jnp.float32),
            pltpu.SemaphoreType.DMA((2,)),
        ],
    )(queries, keys)


def _sc_gather(keys, flat_idx):
    b, d = flat_idx.shape[0], keys.shape[1]
    info = plsc.get_sparse_core_info()
    nw = info.num_cores * info.num_subcores
    bpw = b // nw
    mesh = plsc.VectorSubcoreMesh(core_axis_name="c", subcore_axis_name="s")

    @functools.partial(
        pl.kernel,
        mesh=mesh,
        out_type=jax.ShapeDtypeStruct((b, d), jnp.float32),
        scratch_types=[
            pltpu.VMEM((bpw,), jnp.int32),
            pltpu.VMEM((bpw, d), jnp.float32),
            pltpu.SemaphoreType.DMA,
        ],
    )
    def gather_kernel(keys_hbm, idx_hbm, out_hbm, idx_v, rows_v, sem):
        wid = lax.axis_index("s") * info.num_cores + lax.axis_index("c")
        base = wid * bpw
        pltpu.sync_copy(idx_hbm.at[pl.ds(base, bpw)], idx_v)
        pltpu.async_copy(keys_hbm.at[idx_v], rows_v, sem).wait()
        pltpu.sync_copy(rows_v, out_hbm.at[pl.ds(base, bpw)])

    return gather_kernel(keys, flat_idx)


def kernel(queries, keys, k):
    del k
    topk_dist, topk_idx = _topk_call(queries, keys)
    gathered = _sc_gather(keys, topk_idx.reshape(-1))
    return gathered.reshape(_Q, _TOPK, _D), topk_dist, topk_idx
